# Initial kernel scaffold; baseline (speedup 1.0000x reference)
import functools

import jax
import jax.numpy as jnp
from jax.experimental import pallas as pl
from jax.experimental.pallas import tpu as pltpu

NUM_EXPERTS = 128
TOP_K = 6
HIDDEN = 768
INTER = 1856
T = 4096
P = T * TOP_K
BM = 256
NB = P // BM + NUM_EXPERTS - 1
NB_PAD = NB + 1
M_PAD = NB_PAD * BM


def _routing_metadata(top_k_index):
    e_flat = top_k_index.reshape(-1).astype(jnp.int32)
    order = jnp.argsort(e_flat).astype(jnp.int32)
    sorted_e = e_flat[order]
    counts = jnp.bincount(e_flat, length=NUM_EXPERTS).astype(jnp.int32)
    blocks_per_e = (counts + BM - 1) // BM
    cum_blocks = jnp.cumsum(blocks_per_e).astype(jnp.int32)
    block_start_e = cum_blocks - blocks_per_e
    count_start_e = (jnp.cumsum(counts) - counts).astype(jnp.int32)
    i = jnp.arange(P, dtype=jnp.int32)
    rank = i - count_start_e[sorted_e]
    dest_row = block_start_e[sorted_e] * BM + rank
    tok_sorted = (order // TOP_K).astype(jnp.int32)
    num_used = cum_blocks[-1]
    bidx = jnp.arange(NB, dtype=jnp.int32)
    raw_owner = jnp.minimum(
        jnp.searchsorted(cum_blocks, bidx, side="right"), NUM_EXPERTS - 1
    ).astype(jnp.int32)
    last_owner = jnp.take(raw_owner, num_used - 1)
    block_expert = jnp.where(bidx < num_used, raw_owner, last_owner)
    block_row = jnp.minimum(bidx, num_used - 1)
    block_valid = (bidx < num_used).astype(jnp.int32)
    return order, dest_row, tok_sorted, block_expert, block_row, block_valid


def _gemm_body(be_ref, br_ref, bv_ref, x_ref, wg_ref, wu_ref, wd_ref, w_ref,
               oa_ref, ob_ref):
    b = pl.program_id(0)

    @pl.when(bv_ref[b] == 1)
    def _():
        x = x_ref[...]
        wg = wg_ref[0, :, 0, :]
        wu = wu_ref[0, :, 0, :]
        gate = jnp.dot(x, wg, preferred_element_type=jnp.float32)
        up = jnp.dot(x, wu, preferred_element_type=jnp.float32)
        inter = gate * jax.nn.sigmoid(gate) * up
        out = jnp.dot(inter, wd_ref[0], preferred_element_type=jnp.float32)
        w = w_ref[0, 0, :]
        out = out * w[:, None]
        oa_ref[...] = out[:, : HIDDEN // 2]
        ob_ref[...] = out[:, HIDDEN // 2 :]


def _grouped_mlp(xg, row_w, W_gate_up, W_down, block_expert, block_row,
                 block_valid):
    wgu4 = W_gate_up.reshape(NUM_EXPERTS, HIDDEN, 2, INTER)
    grid_spec = pltpu.PrefetchScalarGridSpec(
        num_scalar_prefetch=3,
        grid=(NB,),
        in_specs=[
            pl.BlockSpec((BM, HIDDEN), lambda b, be, br, bv: (br[b], 0)),
            pl.BlockSpec((1, HIDDEN, 1, INTER), lambda b, be, br, bv: (be[b], 0, 0, 0)),
            pl.BlockSpec((1, HIDDEN, 1, INTER), lambda b, be, br, bv: (be[b], 0, 1, 0)),
            pl.BlockSpec((1, INTER, HIDDEN), lambda b, be, br, bv: (be[b], 0, 0)),
            pl.BlockSpec((1, 1, BM), lambda b, be, br, bv: (br[b], 0, 0)),
        ],
        out_specs=[
            pl.BlockSpec((BM, HIDDEN // 2), lambda b, be, br, bv: (br[b], 0)),
            pl.BlockSpec((BM, HIDDEN // 2), lambda b, be, br, bv: (br[b], 0)),
        ],
    )
    oa, ob = pl.pallas_call(
        _gemm_body,
        grid_spec=grid_spec,
        out_shape=[
            jax.ShapeDtypeStruct((M_PAD, HIDDEN // 2), jnp.float32),
            jax.ShapeDtypeStruct((M_PAD, HIDDEN // 2), jnp.float32),
        ],
        compiler_params=pltpu.CompilerParams(
            dimension_semantics=("arbitrary",),
        ),
    )(block_expert, block_row, block_valid, xg, wgu4, wgu4, W_down, row_w)
    return oa, ob


def kernel(hidden_states, top_k_index, top_k_weights, W_gate_up, W_down):
    (order, dest_row, tok_sorted, block_expert, block_row,
     block_valid) = _routing_metadata(top_k_index)

    w_sorted = top_k_weights.reshape(-1)[order]

    xg = jnp.zeros((M_PAD, HIDDEN), jnp.float32).at[dest_row].set(
        hidden_states[tok_sorted])
    row_w = jnp.zeros((M_PAD,), jnp.float32).at[dest_row].set(w_sorted)
    row_w = row_w.reshape(NB_PAD, 1, BM)

    oa, ob = _grouped_mlp(xg, row_w, W_gate_up, W_down, block_expert,
                          block_row, block_valid)

    out = jnp.concatenate([oa, ob], axis=1)
    final = jnp.zeros((T, HIDDEN), jnp.float32).at[tok_sorted].add(
        out[dest_row])
    return (final, final)


# speedup vs baseline: 4.2918x; 4.2918x over previous
import functools

import jax
import jax.numpy as jnp
from jax.experimental import pallas as pl
from jax.experimental.pallas import tpu as pltpu

NUM_EXPERTS = 128
TOP_K = 6
HIDDEN = 768
INTER = 1856
T = 4096
P = T * TOP_K
BM = 256
NB = P // BM + NUM_EXPERTS - 1
NB_PAD = NB + 1
M_PAD = NB_PAD * BM


def _routing_metadata(top_k_index):
    e_flat = top_k_index.reshape(-1).astype(jnp.int32)
    order = jnp.argsort(e_flat).astype(jnp.int32)
    sorted_e = e_flat[order]
    counts = jnp.bincount(e_flat, length=NUM_EXPERTS).astype(jnp.int32)
    blocks_per_e = (counts + BM - 1) // BM
    cum_blocks = jnp.cumsum(blocks_per_e).astype(jnp.int32)
    block_start_e = cum_blocks - blocks_per_e
    count_start_e = (jnp.cumsum(counts) - counts).astype(jnp.int32)
    i = jnp.arange(P, dtype=jnp.int32)
    rank = i - count_start_e[sorted_e]
    dest_row = block_start_e[sorted_e] * BM + rank
    tok_sorted = (order // TOP_K).astype(jnp.int32)
    num_used = cum_blocks[-1]
    bidx = jnp.arange(NB, dtype=jnp.int32)
    raw_owner = jnp.minimum(
        jnp.searchsorted(cum_blocks, bidx, side="right"), NUM_EXPERTS - 1
    ).astype(jnp.int32)
    last_owner = jnp.take(raw_owner, num_used - 1)
    block_expert = jnp.where(bidx < num_used, raw_owner, last_owner)
    block_row = jnp.minimum(bidx, num_used - 1)
    block_valid = (bidx < num_used).astype(jnp.int32)
    return order, dest_row, tok_sorted, block_expert, block_row, block_valid


def _gemm_body(be_ref, br_ref, bv_ref, x_ref, wgu_ref, wd_ref, w_ref,
               oa_ref, ob_ref):
    b = pl.program_id(0)

    @pl.when(bv_ref[b] == 1)
    def _():
        x = x_ref[...]
        gu = jnp.dot(x, wgu_ref[0], preferred_element_type=jnp.float32)
        gate = gu[:, :INTER]
        up = gu[:, INTER:]
        inter = gate * jax.nn.sigmoid(gate) * up
        out = jnp.dot(inter, wd_ref[0], preferred_element_type=jnp.float32)
        w = w_ref[0, 0, :]
        out = out * w[:, None]
        oa_ref[...] = out[:, : HIDDEN // 2]
        ob_ref[...] = out[:, HIDDEN // 2 :]


def _grouped_mlp(xg, row_w, W_gate_up, W_down, block_expert, block_row,
                 block_valid):
    grid_spec = pltpu.PrefetchScalarGridSpec(
        num_scalar_prefetch=3,
        grid=(NB,),
        in_specs=[
            pl.BlockSpec((BM, HIDDEN), lambda b, be, br, bv: (br[b], 0)),
            pl.BlockSpec((1, HIDDEN, 2 * INTER), lambda b, be, br, bv: (be[b], 0, 0)),
            pl.BlockSpec((1, INTER, HIDDEN), lambda b, be, br, bv: (be[b], 0, 0)),
            pl.BlockSpec((1, 1, BM), lambda b, be, br, bv: (br[b], 0, 0)),
        ],
        out_specs=[
            pl.BlockSpec((BM, HIDDEN // 2), lambda b, be, br, bv: (br[b], 0)),
            pl.BlockSpec((BM, HIDDEN // 2), lambda b, be, br, bv: (br[b], 0)),
        ],
    )
    oa, ob = pl.pallas_call(
        _gemm_body,
        grid_spec=grid_spec,
        out_shape=[
            jax.ShapeDtypeStruct((M_PAD, HIDDEN // 2), jnp.float32),
            jax.ShapeDtypeStruct((M_PAD, HIDDEN // 2), jnp.float32),
        ],
        compiler_params=pltpu.CompilerParams(
            dimension_semantics=("arbitrary",),
        ),
    )(block_expert, block_row, block_valid, xg, W_gate_up, W_down, row_w)
    return oa, ob


def kernel(hidden_states, top_k_index, top_k_weights, W_gate_up, W_down):
    (order, dest_row, tok_sorted, block_expert, block_row,
     block_valid) = _routing_metadata(top_k_index)

    w_sorted = top_k_weights.reshape(-1)[order]

    xg = jnp.zeros((M_PAD, HIDDEN), jnp.float32).at[dest_row].set(
        hidden_states[tok_sorted])
    row_w = jnp.zeros((M_PAD,), jnp.float32).at[dest_row].set(w_sorted)
    row_w = row_w.reshape(NB_PAD, 1, BM)

    oa, ob = _grouped_mlp(xg, row_w, W_gate_up, W_down, block_expert,
                          block_row, block_valid)

    out = jnp.concatenate([oa, ob], axis=1)
    final = jnp.zeros((T, HIDDEN), jnp.float32).at[tok_sorted].add(
        out[dest_row])
    return (final, final)
